# baseline (speedup 1.0000x reference)
import functools

import jax
import jax.numpy as jnp
from jax import lax
from jax.experimental import pallas as pl
from jax.experimental.pallas import tpu as pltpu
from jax.experimental.pallas import tpu_sc as plsc

N_NODES = 10000
N_EDGES = 320000
D = 128

NC = 2
NS = 16
CHUNK = 128
KA = 102
KB = 58
KT = KA + KB
E_PAD = NS * KT * CHUNK
EA = NS * KA * CHUNK
ACC_ROWS = 10112
RPS = ACC_ROWS // NS


def _sc_aggregate(x, idxA, idxB, zeros):
    mesh = plsc.VectorSubcoreMesh(core_axis_name="c", subcore_axis_name="s")

    @functools.partial(
        pl.kernel,
        out_type=jax.ShapeDtypeStruct((NC * ACC_ROWS, D), jnp.float32),
        mesh=mesh,
        scratch_types=[
            pltpu.VMEM((4, 2, CHUNK), jnp.int32),
            pltpu.VMEM((CHUNK, D), jnp.float32),
            pltpu.VMEM((CHUNK, D), jnp.float32),
            pltpu.VMEM_SHARED((ACC_ROWS, D), jnp.float32),
            pltpu.SemaphoreType.DMA,
            pltpu.SemaphoreType.DMA,
            pltpu.SemaphoreType.DMA,
            pltpu.SemaphoreType.DMA,
        ],
    )
    def agg_kernel(x_hbm, idxA_hbm, idxB_hbm, zeros_hbm, out_hbm,
                   ring, rows0, rows1, acc_sh, sem_g, sem_i, sem_s0, sem_s1):
        c = lax.axis_index("c")
        s = lax.axis_index("s")
        kc = lax.select(c == 0, jnp.int32(KA), jnp.int32(KB))
        rows = (rows0, rows1)
        sems = (sem_s0, sem_s1)

        def fetch_idx(j, slot):
            @pl.when(c == 0)
            def _():
                pltpu.async_copy(idxA_hbm.at[s, j], ring.at[slot], sem_i)

            @pl.when(c == 1)
            def _():
                pltpu.async_copy(idxB_hbm.at[s, j], ring.at[slot], sem_i)

        def wait_idx():
            pltpu.make_async_copy(idxA_hbm.at[s, 0], ring.at[0], sem_i).wait()

        def wait_gather(b):
            pltpu.make_async_copy(x_hbm.at[ring.at[0, 0]], rows[b], sem_g).wait()

        def wait_scatter(b):
            pltpu.make_async_copy(rows[b], acc_sh.at[ring.at[0, 1]], sems[b]).wait()

        pltpu.sync_copy(zeros_hbm, acc_sh.at[pl.ds(s * RPS, RPS)])
        plsc.subcore_barrier()

        fetch_idx(0, 0)
        wait_idx()
        fetch_idx(1, 1)
        wait_idx()
        pltpu.async_copy(x_hbm.at[ring.at[0, 0]], rows0, sem_g)
        fetch_idx(2, 2)
        wait_gather(0)
        pltpu.async_copy(rows0, acc_sh.at[ring.at[0, 1]], sem_s0, add=True)
        pltpu.async_copy(x_hbm.at[ring.at[1, 0]], rows1, sem_g)

        def half(j, o):
            b = (1 + o) % 2
            q = (1 + o) % 4
            q1 = (2 + o) % 4
            q2 = (3 + o) % 4
            wait_scatter(1 - b)
            wait_idx()
            fetch_idx(jnp.minimum(j + 2, kc - 1), q2)
            wait_gather(b)
            pltpu.async_copy(rows[b], acc_sh.at[ring.at[q, 1]], sems[b], add=True)
            pltpu.async_copy(x_hbm.at[ring.at[q1, 0]], rows[1 - b], sem_g)

        def body(i, carry):
            j = 4 * i + 1
            half(j, 0)
            half(j + 1, 1)
            half(j + 2, 2)
            half(j + 3, 3)
            return carry

        nt = lax.select(c == 0, jnp.int32((KA - 2) // 4), jnp.int32((KB - 2) // 4))
        lax.fori_loop(0, nt, body, 0)

        wait_scatter(0)
        wait_idx()
        wait_gather(1)
        pltpu.async_copy(rows1, acc_sh.at[ring.at[1, 1]], sem_s1, add=True)
        wait_scatter(1)
        plsc.subcore_barrier()

        row0 = c * ACC_ROWS + s * RPS
        pltpu.sync_copy(acc_sh.at[pl.ds(s * RPS, RPS)], out_hbm.at[pl.ds(row0, RPS)])

    return agg_kernel(x, idxA, idxB, zeros)


def _tc_layer1(aggp, x, WlT, bl, WrT):
    def body(aggp_ref, x_ref, wl_ref, bl_ref, wr_ref, out_ref):
        agg = aggp_ref[:N_NODES, :] + aggp_ref[ACC_ROWS:ACC_ROWS + N_NODES, :]
        r = (jnp.dot(agg, wl_ref[...], preferred_element_type=jnp.float32)
             + bl_ref[...]
             + jnp.dot(x_ref[...], wr_ref[...], preferred_element_type=jnp.float32))
        out_ref[...] = jnp.maximum(r, 0.0)

    return pl.pallas_call(
        body,
        out_shape=jax.ShapeDtypeStruct((N_NODES, D), jnp.float32),
    )(aggp, x, WlT, bl, WrT)


def _tc_layer2(aggp, h, WlT, bl, WrT, WlinT, blin):
    def body(aggp_ref, h_ref, wl_ref, bl_ref, wr_ref, wlin_ref, blin_ref, out_ref):
        agg = aggp_ref[:N_NODES, :] + aggp_ref[ACC_ROWS:ACC_ROWS + N_NODES, :]
        r = (jnp.dot(agg, wl_ref[...], preferred_element_type=jnp.float32)
             + bl_ref[...]
             + jnp.dot(h_ref[...], wr_ref[...], preferred_element_type=jnp.float32))
        h2 = jnp.maximum(r, 0.0)
        out_ref[...] = (jnp.dot(h2, wlin_ref[...], preferred_element_type=jnp.float32)
                        + blin_ref[...])

    return pl.pallas_call(
        body,
        out_shape=jax.ShapeDtypeStruct((N_NODES, D), jnp.float32),
    )(aggp, h, WlT, bl, WrT, WlinT, blin)


def kernel(x, edge_index, Wl1, bl1, Wr1, Wl2, bl2, Wr2, Wlin, blin):
    src = edge_index[0].astype(jnp.int32)
    dst = edge_index[1].astype(jnp.int32)
    pad = E_PAD - N_EDGES
    src_p = jnp.concatenate([src, jnp.zeros((pad,), jnp.int32)])
    dst_p = jnp.concatenate([dst, jnp.full((pad,), N_NODES, jnp.int32)])
    srcA = src_p[:EA].reshape(NS, KA, CHUNK)
    dstA = dst_p[:EA].reshape(NS, KA, CHUNK)
    srcB = src_p[EA:].reshape(NS, KB, CHUNK)
    dstB = dst_p[EA:].reshape(NS, KB, CHUNK)
    idxA = jnp.stack([srcA, dstA], axis=2)
    idxB = jnp.stack([srcB, dstB], axis=2)
    zeros = jnp.zeros((RPS, D), jnp.float32)

    aggp1 = _sc_aggregate(x, idxA, idxB, zeros)
    h1 = _tc_layer1(aggp1, x, Wl1.T, bl1.reshape(1, D), Wr1.T)
    aggp2 = _sc_aggregate(h1, idxA, idxB, zeros)
    out = _tc_layer2(aggp2, h1, Wl2.T, bl2.reshape(1, D), Wr2.T,
                     Wlin.T, blin.reshape(1, D))
    return out


# speedup vs baseline: 1.9017x; 1.9017x over previous
import functools

import jax
import jax.numpy as jnp
from jax import lax
from jax.experimental import pallas as pl
from jax.experimental.pallas import tpu as pltpu
from jax.experimental.pallas import tpu_sc as plsc

N_NODES = 10000
N_EDGES = 320000
D = 128

NC = 2
NS = 16
CHUNK = 128
HALF = CHUNK // 2
KA = 102
KB = 55
KT = KA + KB
KM = max(KA, KB)
E_PAD = NS * KT * CHUNK
EA = NS * KA * CHUNK
ACC_ROWS = 10112
RPS = ACC_ROWS // NS


def _sc_aggregate(x, srcA, dstA, srcB, dstB, zeros):
    mesh = plsc.VectorSubcoreMesh(core_axis_name="c", subcore_axis_name="s")

    @functools.partial(
        pl.kernel,
        out_type=jax.ShapeDtypeStruct((NC * ACC_ROWS, D), jnp.float32),
        mesh=mesh,
        scratch_types=[
            pltpu.VMEM((KM, CHUNK), jnp.int32),
            pltpu.VMEM((KM, CHUNK), jnp.int32),
            pltpu.VMEM((CHUNK, D), jnp.float32),
            pltpu.VMEM_SHARED((ACC_ROWS, D), jnp.float32),
            pltpu.SemaphoreType.DMA,
            pltpu.SemaphoreType.DMA,
            pltpu.SemaphoreType.DMA,
        ],
    )
    def agg_kernel(x_hbm, srcA_hbm, dstA_hbm, srcB_hbm, dstB_hbm, zeros_hbm,
                   out_hbm, src_v, dst_v, rows_v, acc_sh, sem_g, sem_s0, sem_s1):
        c = lax.axis_index("c")
        s = lax.axis_index("s")
        kc = lax.select(c == 0, jnp.int32(KA), jnp.int32(KB))
        halves = (rows_v.at[pl.ds(0, HALF)], rows_v.at[pl.ds(HALF, HALF)])
        sems = (sem_s0, sem_s1)

        @pl.when(c == 0)
        def _():
            pltpu.sync_copy(srcA_hbm.at[s], src_v.at[pl.ds(0, KA)])
            pltpu.sync_copy(dstA_hbm.at[s], dst_v.at[pl.ds(0, KA)])

        @pl.when(c == 1)
        def _():
            pltpu.sync_copy(srcB_hbm.at[s], src_v.at[pl.ds(0, KB)])
            pltpu.sync_copy(dstB_hbm.at[s], dst_v.at[pl.ds(0, KB)])

        pltpu.sync_copy(zeros_hbm, acc_sh.at[pl.ds(s * RPS, RPS)])
        plsc.subcore_barrier()

        def gather(j, u):
            pltpu.async_copy(x_hbm.at[src_v.at[j, pl.ds(u * HALF, HALF)]],
                             halves[u], sem_g)

        def wait_gather(u):
            pltpu.make_async_copy(x_hbm.at[src_v.at[0, pl.ds(0, HALF)]],
                                  halves[u], sem_g).wait()

        def scatter(j, u):
            pltpu.async_copy(halves[u], acc_sh.at[dst_v.at[j, pl.ds(u * HALF, HALF)]],
                             sems[u], add=True)

        def wait_scatter(u):
            pltpu.make_async_copy(halves[u], acc_sh.at[dst_v.at[0, pl.ds(0, HALF)]],
                                  sems[u]).wait()

        gather(0, 0)
        wait_gather(0)
        scatter(0, 0)
        gather(0, 1)

        def body(i, carry):
            wait_gather(1)
            scatter(i, 1)
            wait_scatter(0)
            gather(i + 1, 0)
            wait_gather(0)
            scatter(i + 1, 0)
            wait_scatter(1)
            gather(i + 1, 1)
            return carry

        lax.fori_loop(0, kc - 1, body, 0)
        wait_gather(1)
        scatter(kc - 1, 1)
        wait_scatter(0)
        wait_scatter(1)
        plsc.subcore_barrier()

        row0 = c * ACC_ROWS + s * RPS
        pltpu.sync_copy(acc_sh.at[pl.ds(s * RPS, RPS)], out_hbm.at[pl.ds(row0, RPS)])

    return agg_kernel(x, srcA, dstA, srcB, dstB, zeros)


def _tc_layer1(aggp, x, WlT, bl, WrT):
    def body(aggp_ref, x_ref, wl_ref, bl_ref, wr_ref, out_ref):
        agg = aggp_ref[:N_NODES, :] + aggp_ref[ACC_ROWS:ACC_ROWS + N_NODES, :]
        r = (jnp.dot(agg, wl_ref[...], preferred_element_type=jnp.float32)
             + bl_ref[...]
             + jnp.dot(x_ref[...], wr_ref[...], preferred_element_type=jnp.float32))
        out_ref[...] = jnp.maximum(r, 0.0)

    return pl.pallas_call(
        body,
        out_shape=jax.ShapeDtypeStruct((N_NODES, D), jnp.float32),
    )(aggp, x, WlT, bl, WrT)


def _tc_layer2(aggp, h, WlT, bl, WrT, WlinT, blin):
    def body(aggp_ref, h_ref, wl_ref, bl_ref, wr_ref, wlin_ref, blin_ref, out_ref):
        agg = aggp_ref[:N_NODES, :] + aggp_ref[ACC_ROWS:ACC_ROWS + N_NODES, :]
        r = (jnp.dot(agg, wl_ref[...], preferred_element_type=jnp.float32)
             + bl_ref[...]
             + jnp.dot(h_ref[...], wr_ref[...], preferred_element_type=jnp.float32))
        h2 = jnp.maximum(r, 0.0)
        out_ref[...] = (jnp.dot(h2, wlin_ref[...], preferred_element_type=jnp.float32)
                        + blin_ref[...])

    return pl.pallas_call(
        body,
        out_shape=jax.ShapeDtypeStruct((N_NODES, D), jnp.float32),
    )(aggp, h, WlT, bl, WrT, WlinT, blin)


def kernel(x, edge_index, Wl1, bl1, Wr1, Wl2, bl2, Wr2, Wlin, blin):
    src = edge_index[0].astype(jnp.int32)
    dst = edge_index[1].astype(jnp.int32)
    pad = E_PAD - N_EDGES
    src_p = jnp.concatenate([src, jnp.zeros((pad,), jnp.int32)])
    dst_p = jnp.concatenate([dst, jnp.full((pad,), N_NODES, jnp.int32)])
    srcA = src_p[:EA].reshape(NS, KA, CHUNK)
    dstA = dst_p[:EA].reshape(NS, KA, CHUNK)
    srcB = src_p[EA:].reshape(NS, KB, CHUNK)
    dstB = dst_p[EA:].reshape(NS, KB, CHUNK)
    zeros = jnp.zeros((RPS, D), jnp.float32)

    aggp1 = _sc_aggregate(x, srcA, dstA, srcB, dstB, zeros)
    h1 = _tc_layer1(aggp1, x, Wl1.T, bl1.reshape(1, D), Wr1.T)
    aggp2 = _sc_aggregate(h1, srcA, dstA, srcB, dstB, zeros)
    out = _tc_layer2(aggp2, h1, Wl2.T, bl2.reshape(1, D), Wr2.T,
                     Wlin.T, blin.reshape(1, D))
    return out


# speedup vs baseline: 1.9313x; 1.0156x over previous
import functools

import jax
import jax.numpy as jnp
from jax import lax
from jax.experimental import pallas as pl
from jax.experimental.pallas import tpu as pltpu
from jax.experimental.pallas import tpu_sc as plsc

N_NODES = 10000
N_EDGES = 320000
D = 128

NC = 2
NS = 16
CHUNK = 128
KT = -(-N_EDGES // (NS * CHUNK))
KA = 102
KB = KT - KA
KM = max(KA, KB)
E_PAD = NS * KT * CHUNK
EA = NS * KA * CHUNK
ACC_ROWS = 10112
RPS = ACC_ROWS // NS


def _sc_aggregate(x, srcA, dstA, srcB, dstB, zeros):
    mesh = plsc.VectorSubcoreMesh(core_axis_name="c", subcore_axis_name="s")

    @functools.partial(
        pl.kernel,
        out_type=jax.ShapeDtypeStruct((NC * ACC_ROWS, D), jnp.float32),
        mesh=mesh,
        scratch_types=[
            pltpu.VMEM((KM, CHUNK), jnp.int32),
            pltpu.VMEM((KM, CHUNK), jnp.int32),
            pltpu.VMEM((CHUNK, D), jnp.float32),
            pltpu.VMEM_SHARED((ACC_ROWS, D), jnp.float32),
            pltpu.SemaphoreType.DMA,
        ],
    )
    def agg_kernel(x_hbm, srcA_hbm, dstA_hbm, srcB_hbm, dstB_hbm, zeros_hbm,
                   out_hbm, src_v, dst_v, rows_v, acc_sh, sem):
        c = lax.axis_index("c")
        s = lax.axis_index("s")

        @pl.when(c == 0)
        def _():
            pltpu.sync_copy(srcA_hbm.at[s], src_v.at[pl.ds(0, KA)])
            pltpu.sync_copy(dstA_hbm.at[s], dst_v.at[pl.ds(0, KA)])

        @pl.when(c == 1)
        def _():
            pltpu.sync_copy(srcB_hbm.at[s], src_v.at[pl.ds(0, KB)])
            pltpu.sync_copy(dstB_hbm.at[s], dst_v.at[pl.ds(0, KB)])

        pltpu.sync_copy(zeros_hbm, acc_sh.at[pl.ds(s * RPS, RPS)])
        plsc.subcore_barrier()

        def body(j, carry):
            pltpu.async_copy(x_hbm.at[src_v.at[j]], rows_v, sem).wait()
            pltpu.sync_copy(rows_v, acc_sh.at[dst_v.at[j]], add=True)
            return carry

        kc = lax.select(c == 0, jnp.int32(KA), jnp.int32(KB))
        lax.fori_loop(0, kc, body, 0)
        plsc.subcore_barrier()

        row0 = c * ACC_ROWS + s * RPS
        pltpu.sync_copy(acc_sh.at[pl.ds(s * RPS, RPS)], out_hbm.at[pl.ds(row0, RPS)])

    return agg_kernel(x, srcA, dstA, srcB, dstB, zeros)


def _tc_layer1(aggp, x, WlT, bl, WrT):
    def body(aggp_ref, x_ref, wl_ref, bl_ref, wr_ref, out_ref):
        agg = aggp_ref[:N_NODES, :] + aggp_ref[ACC_ROWS:ACC_ROWS + N_NODES, :]
        r = (jnp.dot(agg, wl_ref[...], preferred_element_type=jnp.float32)
             + bl_ref[...]
             + jnp.dot(x_ref[...], wr_ref[...], preferred_element_type=jnp.float32))
        out_ref[...] = jnp.maximum(r, 0.0)

    return pl.pallas_call(
        body,
        out_shape=jax.ShapeDtypeStruct((N_NODES, D), jnp.float32),
    )(aggp, x, WlT, bl, WrT)


def _tc_layer2(aggp, h, WlT, bl, WrT, WlinT, blin):
    def body(aggp_ref, h_ref, wl_ref, bl_ref, wr_ref, wlin_ref, blin_ref, out_ref):
        agg = aggp_ref[:N_NODES, :] + aggp_ref[ACC_ROWS:ACC_ROWS + N_NODES, :]
        r = (jnp.dot(agg, wl_ref[...], preferred_element_type=jnp.float32)
             + bl_ref[...]
             + jnp.dot(h_ref[...], wr_ref[...], preferred_element_type=jnp.float32))
        h2 = jnp.maximum(r, 0.0)
        out_ref[...] = (jnp.dot(h2, wlin_ref[...], preferred_element_type=jnp.float32)
                        + blin_ref[...])

    return pl.pallas_call(
        body,
        out_shape=jax.ShapeDtypeStruct((N_NODES, D), jnp.float32),
    )(aggp, h, WlT, bl, WrT, WlinT, blin)


def kernel(x, edge_index, Wl1, bl1, Wr1, Wl2, bl2, Wr2, Wlin, blin):
    src = edge_index[0].astype(jnp.int32)
    dst = edge_index[1].astype(jnp.int32)
    pad = E_PAD - N_EDGES
    src_p = jnp.concatenate([src, jnp.zeros((pad,), jnp.int32)])
    dst_p = jnp.concatenate([dst, jnp.full((pad,), N_NODES, jnp.int32)])
    srcA = src_p[:EA].reshape(NS, KA, CHUNK)
    dstA = dst_p[:EA].reshape(NS, KA, CHUNK)
    srcB = src_p[EA:].reshape(NS, KB, CHUNK)
    dstB = dst_p[EA:].reshape(NS, KB, CHUNK)
    zeros = jnp.zeros((RPS, D), jnp.float32)

    aggp1 = _sc_aggregate(x, srcA, dstA, srcB, dstB, zeros)
    h1 = _tc_layer1(aggp1, x, Wl1.T, bl1.reshape(1, D), Wr1.T)
    aggp2 = _sc_aggregate(h1, srcA, dstA, srcB, dstB, zeros)
    out = _tc_layer2(aggp2, h1, Wl2.T, bl2.reshape(1, D), Wr2.T,
                     Wlin.T, blin.reshape(1, D))
    return out
